# baseline (speedup 1.0000x reference)
import functools

import jax
import jax.numpy as jnp
from jax import lax
from jax.experimental import pallas as pl
from jax.experimental.pallas import tpu as pltpu
from jax.experimental.pallas import tpu_sc as plsc

N, A, B, C = 128, 64, 64, 64
BATCH = 4096
TOTAL = N * BATCH
LANES = 16

_info = plsc.get_sparse_core_info()
NC = _info.num_cores
NS = _info.num_subcores
NW = NC * NS
PER_W = TOTAL // NW
CHUNK = 128
N_CHUNKS = PER_W // CHUNK
VEC_PER_ROW = BATCH // LANES
ROWS_PER_W = N // NW


def _sc_body(table, a_h, b_h, c_h, out, a_v, b_v, c_v, idx_v, res_v, sem):
    wid = lax.axis_index("s") * NC + lax.axis_index("c")
    base = wid * PER_W

    pltpu.sync_copy(a_h.at[pl.ds(base, PER_W)], a_v)
    pltpu.sync_copy(b_h.at[pl.ds(base, PER_W)], b_v)
    pltpu.sync_copy(c_h.at[pl.ds(base, PER_W)], c_v)

    row0 = wid * ROWS_PER_W

    def idx_body(j, _):
        s = pl.ds(j * LANES, LANES)
        row = row0 + j // VEC_PER_ROW
        hi = jnp.full((LANES,), row << 18, jnp.int32)
        idx_v[s] = hi | (a_v[s] << 12) | (b_v[s] << 6) | c_v[s]
        return 0

    lax.fori_loop(0, PER_W // LANES, idx_body, 0)

    def fire(g, _):
        s = pl.ds(g * CHUNK, CHUNK)
        pltpu.async_copy(table.at[idx_v.at[s]], res_v.at[s], sem)
        return 0

    lax.fori_loop(0, N_CHUNKS, fire, 0)

    def drain(g, _):
        s = pl.ds(0, CHUNK)
        pltpu.make_async_copy(table.at[idx_v.at[s]], res_v.at[s], sem).wait()
        return 0

    lax.fori_loop(0, N_CHUNKS, drain, 0)

    pltpu.sync_copy(res_v, out.at[pl.ds(base, PER_W)])


@jax.jit
def _sc_gather(table, a_f, b_f, c_f):
    mesh = plsc.VectorSubcoreMesh(core_axis_name="c", subcore_axis_name="s")
    return pl.kernel(
        _sc_body,
        mesh=mesh,
        out_type=jax.ShapeDtypeStruct((TOTAL,), jnp.float32),
        scratch_types=[
            pltpu.VMEM((PER_W,), jnp.int32),
            pltpu.VMEM((PER_W,), jnp.int32),
            pltpu.VMEM((PER_W,), jnp.int32),
            pltpu.VMEM((PER_W,), jnp.int32),
            pltpu.VMEM((PER_W,), jnp.float32),
            pltpu.SemaphoreType.DMA,
        ],
    )(table, a_f, b_f, c_f)


def kernel(joint, a, b, c):
    table = joint.reshape(-1)
    a_f = a.reshape(-1).astype(jnp.int32)
    b_f = b.reshape(-1).astype(jnp.int32)
    c_f = c.reshape(-1).astype(jnp.int32)
    out = _sc_gather(table, a_f, b_f, c_f)
    return out.reshape(N, BATCH)
